# Initial kernel scaffold; baseline (speedup 1.0000x reference)
import jax
import jax.numpy as jnp
from jax.experimental import pallas as pl


def kernel(node_attr, edge_index, edge_attr, edge_sh, fc_w1, fc_b1, fc_w2, fc_b2, bn_weight, bn_bias):
    raise NotImplementedError("write your pallas kernel here")


# speedup vs baseline: 3.1356x; 3.1356x over previous
import functools

import numpy as np
import jax
import jax.numpy as jnp
from jax import lax
from jax.experimental import pallas as pl
from jax.experimental.pallas import tpu as pltpu
from jax.experimental.pallas import tpu_sc as plsc

N = 50000
E = 800000
NS = 16
NV = 4
SH_DIM = 9
WN = NS * NS + NS * NV
NEF = 3 * NS
TPW = 32
ALPHA = 1.0 / np.sqrt(NS)

NC = 2
NSUB = 16
NW = NC * NSUB
CHUNK = 128
ROWS = E // CHUNK
RLO = ROWS // NW
REM = ROWS % NW
NPT = 3128
NPT_LAST = N - 15 * NPT

_P_np = np.zeros((NS, WN), np.float32)
_S_np = np.zeros((WN, 28), np.float32)
_Q_np = np.zeros((SH_DIM, 28), np.float32)
for _i in range(NS):
    for _j in range(NS):
        _P_np[_i, _i * NS + _j] = 1.0
        _S_np[_i * NS + _j, _j] = 1.0
    for _jv in range(NV):
        _P_np[_i, NS * NS + _i * NV + _jv] = 1.0
        for _d in range(3):
            _S_np[NS * NS + _i * NV + _jv, 16 + 3 * _jv + _d] = 1.0
_Q_np[0, :16] = 1.0
for _jv in range(NV):
    for _d in range(3):
        _Q_np[1 + _d, 16 + 3 * _jv + _d] = 1.0
_R_np = np.zeros((12, NV), np.float32)
_RT_np = np.zeros((NV, 12), np.float32)
for _jv in range(NV):
    for _d in range(3):
        _R_np[3 * _jv + _d, _jv] = 1.0
        _RT_np[_jv, 3 * _jv + _d] = 1.0

_sc_mesh = plsc.VectorSubcoreMesh(core_axis_name="c", subcore_axis_name="s")


def _none(_):
    return None


@functools.partial(
    pl.kernel,
    out_type=jax.ShapeDtypeStruct((E, NS), jnp.float32),
    mesh=_sc_mesh,
    scratch_types=[
        pltpu.VMEM((CHUNK,), jnp.int32),
        pltpu.VMEM((CHUNK, NS), jnp.float32),
        pltpu.SemaphoreType.DMA,
    ],
    compiler_params=pltpu.CompilerParams(use_tc_tiling_on_sc=False),
)
def _sc_gather(tbl_hbm, idx_hbm, out_hbm, idx_v, row_v, sem):
    c = lax.axis_index("c")
    s = lax.axis_index("s")
    wid = s * NC + c
    base = wid * RLO + jnp.minimum(wid, REM)

    def body(j, carry):
        r = base + j
        pltpu.sync_copy(idx_hbm.at[pl.ds(r * CHUNK, CHUNK)], idx_v)
        pltpu.async_copy(tbl_hbm.at[idx_v], row_v, sem).wait()
        pltpu.sync_copy(row_v, out_hbm.at[pl.ds(r * CHUNK, CHUNK)])
        return carry

    lax.fori_loop(0, RLO, body, 0)
    pl.when(wid < REM)(lambda: _none(body(RLO, 0)))


@functools.partial(
    pl.kernel,
    out_type=jax.ShapeDtypeStruct((NC, N, TPW), jnp.float32),
    mesh=_sc_mesh,
    scratch_types=[
        pltpu.VMEM((CHUNK,), jnp.int32),
        pltpu.VMEM((CHUNK, TPW), jnp.float32),
        pltpu.VMEM_SHARED((N, TPW), jnp.float32),
        pltpu.SemaphoreType.DMA,
    ],
    compiler_params=pltpu.CompilerParams(use_tc_tiling_on_sc=False),
)
def _sc_scatter(tp_hbm, idx_hbm, zeros_hbm, out_hbm, idx_v, src_v, acc_sh, sem):
    c = lax.axis_index("c")
    s = lax.axis_index("s")
    wid = s * NC + c
    base = wid * RLO + jnp.minimum(wid, REM)
    pl.when(s < NSUB - 1)(lambda: pltpu.sync_copy(
        zeros_hbm, acc_sh.at[pl.ds(s * NPT, NPT)]))
    pl.when(s == NSUB - 1)(lambda: pltpu.sync_copy(
        zeros_hbm.at[pl.ds(0, NPT_LAST)], acc_sh.at[pl.ds(15 * NPT, NPT_LAST)]))
    plsc.subcore_barrier()

    def body(j, carry):
        r = base + j
        pltpu.sync_copy(idx_hbm.at[pl.ds(r * CHUNK, CHUNK)], idx_v)
        pltpu.sync_copy(tp_hbm.at[pl.ds(r * CHUNK, CHUNK)], src_v)
        pltpu.sync_copy(src_v, acc_sh.at[idx_v], add=True)
        return carry

    lax.fori_loop(0, RLO, body, 0)
    pl.when(wid < REM)(lambda: _none(body(RLO, 0)))
    plsc.subcore_barrier()
    pl.when(s < NSUB - 1)(lambda: pltpu.sync_copy(
        acc_sh.at[pl.ds(s * NPT, NPT)], out_hbm.at[c, pl.ds(s * NPT, NPT)]))
    pl.when(s == NSUB - 1)(lambda: pltpu.sync_copy(
        acc_sh.at[pl.ds(15 * NPT, NPT_LAST)],
        out_hbm.at[c, pl.ds(15 * NPT, NPT_LAST)]))


TILE = 2000
GRID = E // TILE


def _edge_body(ea_ref, sh_ref, x_ref, w1_ref, b1_ref, w2_ref, b2_ref,
               p_ref, s_ref, q_ref, out_ref):
    h = jnp.maximum(
        jnp.dot(ea_ref[...], w1_ref[...], preferred_element_type=jnp.float32)
        + b1_ref[...], 0.0)
    w = jnp.dot(h, w2_ref[...], preferred_element_type=jnp.float32) + b2_ref[...]
    y = jnp.dot(x_ref[...], p_ref[...], preferred_element_type=jnp.float32) * w
    pre = jnp.dot(y, s_ref[...], preferred_element_type=jnp.float32)
    shm = jnp.dot(sh_ref[...], q_ref[...], preferred_element_type=jnp.float32)
    tp = pre * shm * ALPHA
    ones = jnp.ones((TILE, 1), jnp.float32)
    zer = jnp.zeros((TILE, 3), jnp.float32)
    out_ref[...] = jnp.concatenate([tp, ones, zer], axis=1)


_edge_call = pl.pallas_call(
    _edge_body,
    grid=(GRID,),
    in_specs=[
        pl.BlockSpec((TILE, NEF), lambda i: (i, 0)),
        pl.BlockSpec((TILE, SH_DIM), lambda i: (i, 0)),
        pl.BlockSpec((TILE, NS), lambda i: (i, 0)),
        pl.BlockSpec((NEF, NEF), lambda i: (0, 0)),
        pl.BlockSpec((1, NEF), lambda i: (0, 0)),
        pl.BlockSpec((NEF, WN), lambda i: (0, 0)),
        pl.BlockSpec((1, WN), lambda i: (0, 0)),
        pl.BlockSpec((NS, WN), lambda i: (0, 0)),
        pl.BlockSpec((WN, 28), lambda i: (0, 0)),
        pl.BlockSpec((SH_DIM, 28), lambda i: (0, 0)),
    ],
    out_specs=pl.BlockSpec((TILE, TPW), lambda i: (i, 0)),
    out_shape=jax.ShapeDtypeStruct((E, TPW), jnp.float32),
)


NT = 2000
NG = N // NT


def _stats_body(part_ref, out_ref):
    p = part_ref[0] + part_ref[1]
    cnt = jnp.maximum(p[:, 28:29], 1.0)
    o = p / cnt
    sums = jnp.sum(o, axis=0, keepdims=True)
    sq = jnp.sum(o * o, axis=0, keepdims=True)
    blk = jnp.concatenate([sums, sq], axis=0)

    @pl.when(pl.program_id(0) == 0)
    def _():
        out_ref[...] = jnp.zeros_like(out_ref)

    out_ref[...] += blk


_stats_call = pl.pallas_call(
    _stats_body,
    grid=(NG,),
    in_specs=[pl.BlockSpec((2, NT, TPW), lambda i: (0, i, 0))],
    out_specs=pl.BlockSpec((2, TPW), lambda i: (0, 0)),
    out_shape=jax.ShapeDtypeStruct((2, TPW), jnp.float32),
)


def _bn_body(part_ref, stats_ref, bnws_ref, bnwv_ref, bias_ref, r_ref, rt_ref,
             out_ref):
    p = part_ref[0] + part_ref[1]
    cnt = jnp.maximum(p[:, 28:29], 1.0)
    o = p[:, :28] / cnt
    sc = o[:, :NS]
    v = o[:, NS:28]
    stats = stats_ref[...] * (1.0 / N)
    m = stats[0:1, :NS]
    var = stats[1:2, :NS] - m * m
    s_bn = (sc - m) * lax.rsqrt(var + 1e-5) * bnws_ref[...] + bias_ref[...]
    vn = jnp.dot(stats[1:2, NS:28], r_ref[...],
                 preferred_element_type=jnp.float32)
    scl = bnwv_ref[...] * lax.rsqrt(vn + 1e-5)
    v_bn = v * jnp.dot(scl, rt_ref[...], preferred_element_type=jnp.float32)
    out_ref[...] = jnp.concatenate([s_bn, v_bn], axis=1)


_bn_call = pl.pallas_call(
    _bn_body,
    grid=(NG,),
    in_specs=[
        pl.BlockSpec((2, NT, TPW), lambda i: (0, i, 0)),
        pl.BlockSpec((2, TPW), lambda i: (0, 0)),
        pl.BlockSpec((1, NS), lambda i: (0, 0)),
        pl.BlockSpec((1, NV), lambda i: (0, 0)),
        pl.BlockSpec((1, NS), lambda i: (0, 0)),
        pl.BlockSpec((12, NV), lambda i: (0, 0)),
        pl.BlockSpec((NV, 12), lambda i: (0, 0)),
    ],
    out_specs=pl.BlockSpec((NT, 28), lambda i: (i, 0)),
    out_shape=jax.ShapeDtypeStruct((N, 28), jnp.float32),
)


def kernel(node_attr, edge_index, edge_attr, edge_sh, fc_w1, fc_b1, fc_w2,
           fc_b2, bn_weight, bn_bias):
    dst1 = edge_index[1]
    src1 = edge_index[0]
    x = _sc_gather(node_attr, dst1)
    tp = _edge_call(edge_attr, edge_sh, x, fc_w1, fc_b1.reshape(1, NEF),
                    fc_w2, fc_b2.reshape(1, WN), jnp.asarray(_P_np),
                    jnp.asarray(_S_np), jnp.asarray(_Q_np))
    parts = _sc_scatter(tp, src1, jnp.zeros((NPT, TPW), jnp.float32))
    stats = _stats_call(parts)
    out = _bn_call(parts, stats, bn_weight[:NS].reshape(1, NS),
                   bn_weight[NS:].reshape(1, NV), bn_bias.reshape(1, NS),
                   jnp.asarray(_R_np), jnp.asarray(_RT_np))
    return out


# speedup vs baseline: 3.4183x; 1.0901x over previous
import functools

import numpy as np
import jax
import jax.numpy as jnp
from jax import lax
from jax.experimental import pallas as pl
from jax.experimental.pallas import tpu as pltpu
from jax.experimental.pallas import tpu_sc as plsc

N = 50000
E = 800000
NS = 16
NV = 4
SH_DIM = 9
WN = NS * NS + NS * NV
NEF = 3 * NS
TPW = 32
ALPHA = 1.0 / np.sqrt(NS)

NC = 2
NSUB = 16
NW = NC * NSUB
CHUNK = 128
ROWS = E // CHUNK
RLO = ROWS // NW
REM = ROWS % NW
NPT = 3128
NPT_LAST = N - 15 * NPT

_P_np = np.zeros((NS, WN), np.float32)
_S_np = np.zeros((WN, 28), np.float32)
_Q_np = np.zeros((SH_DIM, 28), np.float32)
for _i in range(NS):
    for _j in range(NS):
        _P_np[_i, _i * NS + _j] = 1.0
        _S_np[_i * NS + _j, _j] = 1.0
    for _jv in range(NV):
        _P_np[_i, NS * NS + _i * NV + _jv] = 1.0
        for _d in range(3):
            _S_np[NS * NS + _i * NV + _jv, 16 + 3 * _jv + _d] = 1.0
_Q_np[0, :16] = 1.0
for _jv in range(NV):
    for _d in range(3):
        _Q_np[1 + _d, 16 + 3 * _jv + _d] = 1.0
_R_np = np.zeros((12, NV), np.float32)
_RT_np = np.zeros((NV, 12), np.float32)
for _jv in range(NV):
    for _d in range(3):
        _R_np[3 * _jv + _d, _jv] = 1.0
        _RT_np[_jv, 3 * _jv + _d] = 1.0

_sc_mesh = plsc.VectorSubcoreMesh(core_axis_name="c", subcore_axis_name="s")


def _none(_):
    return None


KG = 13
SUPG = RLO // KG
EG = KG * CHUNK
EW = RLO * CHUNK


@functools.partial(
    pl.kernel,
    out_type=jax.ShapeDtypeStruct((E, NS), jnp.float32),
    mesh=_sc_mesh,
    scratch_types=[
        pltpu.VMEM((EW + CHUNK,), jnp.int32),
        pltpu.VMEM((2, EG, NS), jnp.float32),
        pltpu.SemaphoreType.DMA,
        pltpu.SemaphoreType.DMA,
        pltpu.SemaphoreType.DMA,
        pltpu.SemaphoreType.DMA,
    ],
    compiler_params=pltpu.CompilerParams(use_tc_tiling_on_sc=False),
)
def _sc_gather(tbl_hbm, idx_hbm, out_hbm, idx_v, row_v, g0, g1, s0, s1):
    c = lax.axis_index("c")
    s = lax.axis_index("s")
    wid = s * NC + c
    base = (wid * RLO + jnp.minimum(wid, REM)) * CHUNK
    gsem = (g0, g1)
    ssem = (s0, s1)
    pltpu.sync_copy(idx_hbm.at[pl.ds(base, EW)], idx_v.at[pl.ds(0, EW)])
    pl.when(wid < REM)(lambda: pltpu.sync_copy(
        idx_hbm.at[pl.ds(base + EW, CHUNK)],
        idx_v.at[pl.ds(EW, CHUNK)]))

    def issue_gather(g, b):
        return pltpu.async_copy(
            tbl_hbm.at[idx_v.at[pl.ds(g * EG, EG)]], row_v.at[b], gsem[b])

    def issue_store(g, b):
        return pltpu.async_copy(
            row_v.at[b], out_hbm.at[pl.ds(base + g * EG, EG)], ssem[b])

    stores = [None] * SUPG
    pending = issue_gather(0, 0)
    for g in range(SUPG):
        b = g % 2
        pending.wait()
        stores[g] = issue_store(g, b)
        if g + 1 < SUPG:
            if g >= 1:
                stores[g - 1].wait()
            pending = issue_gather(g + 1, (g + 1) % 2)
    stores[SUPG - 2].wait()
    stores[SUPG - 1].wait()

    def extra():
        pltpu.async_copy(tbl_hbm.at[idx_v.at[pl.ds(EW, CHUNK)]],
                         row_v.at[0, pl.ds(0, CHUNK)], g0).wait()
        pltpu.async_copy(row_v.at[0, pl.ds(0, CHUNK)],
                         out_hbm.at[pl.ds(base + EW, CHUNK)], s0).wait()

    pl.when(wid < REM)(extra)


KS = 3
SUPS = RLO // KS
ES = KS * CHUNK


@functools.partial(
    pl.kernel,
    out_type=jax.ShapeDtypeStruct((NC, N, TPW), jnp.float32),
    mesh=_sc_mesh,
    scratch_types=[
        pltpu.VMEM((2, ES), jnp.int32),
        pltpu.VMEM((2, ES, TPW), jnp.float32),
        pltpu.VMEM_SHARED((N, TPW), jnp.float32),
        pltpu.SemaphoreType.DMA,
        pltpu.SemaphoreType.DMA,
        pltpu.SemaphoreType.DMA,
        pltpu.SemaphoreType.DMA,
    ],
    compiler_params=pltpu.CompilerParams(use_tc_tiling_on_sc=False),
)
def _sc_scatter(tp_hbm, idx_hbm, zeros_hbm, out_hbm, idx_v, src_v, acc_sh,
                g0, g1, s0, s1):
    c = lax.axis_index("c")
    s = lax.axis_index("s")
    wid = s * NC + c
    base = (wid * RLO + jnp.minimum(wid, REM)) * CHUNK
    lsem = (g0, g1)
    asem = (s0, s1)
    pl.when(s < NSUB - 1)(lambda: pltpu.sync_copy(
        zeros_hbm, acc_sh.at[pl.ds(s * NPT, NPT)]))
    pl.when(s == NSUB - 1)(lambda: pltpu.sync_copy(
        zeros_hbm.at[pl.ds(0, NPT_LAST)], acc_sh.at[pl.ds(15 * NPT, NPT_LAST)]))
    plsc.subcore_barrier()

    def issue_load(g, b):
        d1 = pltpu.async_copy(idx_hbm.at[pl.ds(base + g * ES, ES)],
                              idx_v.at[b], lsem[b])
        d2 = pltpu.async_copy(tp_hbm.at[pl.ds(base + g * ES, ES)],
                              src_v.at[b], lsem[b])
        return (d1, d2)

    def issue_scat(g, b):
        return pltpu.async_copy(src_v.at[b], acc_sh.at[idx_v.at[b]], asem[b],
                                add=True)

    scats = [None] * SUPS
    pending = issue_load(0, 0)
    for g in range(SUPS):
        b = g % 2
        pending[0].wait()
        pending[1].wait()
        scats[g] = issue_scat(g, b)
        if g + 1 < SUPS:
            if g >= 1:
                scats[g - 1].wait()
            pending = issue_load(g + 1, (g + 1) % 2)
    scats[SUPS - 2].wait()
    scats[SUPS - 1].wait()

    def extra():
        r = base + RLO * CHUNK
        pltpu.async_copy(idx_hbm.at[pl.ds(r, CHUNK)],
                         idx_v.at[0, pl.ds(0, CHUNK)], g0).wait()
        pltpu.async_copy(tp_hbm.at[pl.ds(r, CHUNK)],
                         src_v.at[0, pl.ds(0, CHUNK)], g0).wait()
        pltpu.async_copy(src_v.at[0, pl.ds(0, CHUNK)],
                         acc_sh.at[idx_v.at[0, pl.ds(0, CHUNK)]], s0,
                         add=True).wait()

    pl.when(wid < REM)(extra)
    plsc.subcore_barrier()
    pl.when(s < NSUB - 1)(lambda: pltpu.sync_copy(
        acc_sh.at[pl.ds(s * NPT, NPT)], out_hbm.at[c, pl.ds(s * NPT, NPT)]))
    pl.when(s == NSUB - 1)(lambda: pltpu.sync_copy(
        acc_sh.at[pl.ds(15 * NPT, NPT_LAST)],
        out_hbm.at[c, pl.ds(15 * NPT, NPT_LAST)]))


TILE = 2000
GRID = E // TILE


def _edge_body(ea_ref, sh_ref, x_ref, w1_ref, b1_ref, w2_ref, b2_ref,
               p_ref, s_ref, q_ref, out_ref):
    h = jnp.maximum(
        jnp.dot(ea_ref[...], w1_ref[...], preferred_element_type=jnp.float32)
        + b1_ref[...], 0.0)
    w = jnp.dot(h, w2_ref[...], preferred_element_type=jnp.float32) + b2_ref[...]
    y = jnp.dot(x_ref[...], p_ref[...], preferred_element_type=jnp.float32) * w
    pre = jnp.dot(y, s_ref[...], preferred_element_type=jnp.float32)
    shm = jnp.dot(sh_ref[...], q_ref[...], preferred_element_type=jnp.float32)
    tp = pre * shm * ALPHA
    ones = jnp.ones((TILE, 1), jnp.float32)
    zer = jnp.zeros((TILE, 3), jnp.float32)
    out_ref[...] = jnp.concatenate([tp, ones, zer], axis=1)


_edge_call = pl.pallas_call(
    _edge_body,
    grid=(GRID,),
    in_specs=[
        pl.BlockSpec((TILE, NEF), lambda i: (i, 0)),
        pl.BlockSpec((TILE, SH_DIM), lambda i: (i, 0)),
        pl.BlockSpec((TILE, NS), lambda i: (i, 0)),
        pl.BlockSpec((NEF, NEF), lambda i: (0, 0)),
        pl.BlockSpec((1, NEF), lambda i: (0, 0)),
        pl.BlockSpec((NEF, WN), lambda i: (0, 0)),
        pl.BlockSpec((1, WN), lambda i: (0, 0)),
        pl.BlockSpec((NS, WN), lambda i: (0, 0)),
        pl.BlockSpec((WN, 28), lambda i: (0, 0)),
        pl.BlockSpec((SH_DIM, 28), lambda i: (0, 0)),
    ],
    out_specs=pl.BlockSpec((TILE, TPW), lambda i: (i, 0)),
    out_shape=jax.ShapeDtypeStruct((E, TPW), jnp.float32),
)


NT = 2000
NG = N // NT


def _stats_body(part_ref, out_ref):
    p = part_ref[0] + part_ref[1]
    cnt = jnp.maximum(p[:, 28:29], 1.0)
    o = p / cnt
    sums = jnp.sum(o, axis=0, keepdims=True)
    sq = jnp.sum(o * o, axis=0, keepdims=True)
    blk = jnp.concatenate([sums, sq], axis=0)

    @pl.when(pl.program_id(0) == 0)
    def _():
        out_ref[...] = jnp.zeros_like(out_ref)

    out_ref[...] += blk


_stats_call = pl.pallas_call(
    _stats_body,
    grid=(NG,),
    in_specs=[pl.BlockSpec((2, NT, TPW), lambda i: (0, i, 0))],
    out_specs=pl.BlockSpec((2, TPW), lambda i: (0, 0)),
    out_shape=jax.ShapeDtypeStruct((2, TPW), jnp.float32),
)


def _bn_body(part_ref, stats_ref, bnws_ref, bnwv_ref, bias_ref, r_ref, rt_ref,
             out_ref):
    p = part_ref[0] + part_ref[1]
    cnt = jnp.maximum(p[:, 28:29], 1.0)
    o = p[:, :28] / cnt
    sc = o[:, :NS]
    v = o[:, NS:28]
    stats = stats_ref[...] * (1.0 / N)
    m = stats[0:1, :NS]
    var = stats[1:2, :NS] - m * m
    s_bn = (sc - m) * lax.rsqrt(var + 1e-5) * bnws_ref[...] + bias_ref[...]
    vn = jnp.dot(stats[1:2, NS:28], r_ref[...],
                 preferred_element_type=jnp.float32)
    scl = bnwv_ref[...] * lax.rsqrt(vn + 1e-5)
    v_bn = v * jnp.dot(scl, rt_ref[...], preferred_element_type=jnp.float32)
    out_ref[...] = jnp.concatenate([s_bn, v_bn], axis=1)


_bn_call = pl.pallas_call(
    _bn_body,
    grid=(NG,),
    in_specs=[
        pl.BlockSpec((2, NT, TPW), lambda i: (0, i, 0)),
        pl.BlockSpec((2, TPW), lambda i: (0, 0)),
        pl.BlockSpec((1, NS), lambda i: (0, 0)),
        pl.BlockSpec((1, NV), lambda i: (0, 0)),
        pl.BlockSpec((1, NS), lambda i: (0, 0)),
        pl.BlockSpec((12, NV), lambda i: (0, 0)),
        pl.BlockSpec((NV, 12), lambda i: (0, 0)),
    ],
    out_specs=pl.BlockSpec((NT, 28), lambda i: (i, 0)),
    out_shape=jax.ShapeDtypeStruct((N, 28), jnp.float32),
)


def kernel(node_attr, edge_index, edge_attr, edge_sh, fc_w1, fc_b1, fc_w2,
           fc_b2, bn_weight, bn_bias):
    dst1 = edge_index[1]
    src1 = edge_index[0]
    x = _sc_gather(node_attr, dst1)
    tp = _edge_call(edge_attr, edge_sh, x, fc_w1, fc_b1.reshape(1, NEF),
                    fc_w2, fc_b2.reshape(1, WN), jnp.asarray(_P_np),
                    jnp.asarray(_S_np), jnp.asarray(_Q_np))
    parts = _sc_scatter(tp, src1, jnp.zeros((NPT, TPW), jnp.float32))
    stats = _stats_call(parts)
    out = _bn_call(parts, stats, bn_weight[:NS].reshape(1, NS),
                   bn_weight[NS:].reshape(1, NV), bn_bias.reshape(1, NS),
                   jnp.asarray(_R_np), jnp.asarray(_RT_np))
    return out


# speedup vs baseline: 3.5700x; 1.0444x over previous
import functools

import numpy as np
import jax
import jax.numpy as jnp
from jax import lax
from jax.experimental import pallas as pl
from jax.experimental.pallas import tpu as pltpu
from jax.experimental.pallas import tpu_sc as plsc

N = 50000
E = 800000
NS = 16
NV = 4
SH_DIM = 9
WN = NS * NS + NS * NV
NEF = 3 * NS
TPW = 32
ALPHA = 1.0 / np.sqrt(NS)

NC = 2
NSUB = 16
NW = NC * NSUB
CHUNK = 128
ROWS = E // CHUNK
RLO = ROWS // NW
REM = ROWS % NW
NPT = 3128
NPT_LAST = N - 15 * NPT

_CHUNKS = [(0, 75, 0), (2400, 75, 0), (4800, 45, 10)]

_P_np = np.zeros((NS, WN), np.float32)
_S_np = np.zeros((WN, 28), np.float32)
_Q_np = np.zeros((SH_DIM, 28), np.float32)
for _i in range(NS):
    for _j in range(NS):
        _P_np[_i, _i * NS + _j] = 1.0
        _S_np[_i * NS + _j, _j] = 1.0
    for _jv in range(NV):
        _P_np[_i, NS * NS + _i * NV + _jv] = 1.0
        for _d in range(3):
            _S_np[NS * NS + _i * NV + _jv, 16 + 3 * _jv + _d] = 1.0
_Q_np[0, :16] = 1.0
for _jv in range(NV):
    for _d in range(3):
        _Q_np[1 + _d, 16 + 3 * _jv + _d] = 1.0
_R_np = np.zeros((12, NV), np.float32)
_RT_np = np.zeros((NV, 12), np.float32)
for _jv in range(NV):
    for _d in range(3):
        _R_np[3 * _jv + _d, _jv] = 1.0
        _RT_np[_jv, 3 * _jv + _d] = 1.0

_sc_mesh = plsc.VectorSubcoreMesh(core_axis_name="c", subcore_axis_name="s")


def _make_gather(start_row, rlo, rem, kg):
    supg = rlo // kg
    eg = kg * CHUNK
    ew = rlo * CHUNK
    ec = (32 * rlo + rem) * CHUNK

    @functools.partial(
        pl.kernel,
        out_type=jax.ShapeDtypeStruct((ec, NS), jnp.float32),
        mesh=_sc_mesh,
        scratch_types=[
            pltpu.VMEM((ew + CHUNK,), jnp.int32),
            pltpu.VMEM((2, eg, NS), jnp.float32),
            pltpu.SemaphoreType.DMA,
            pltpu.SemaphoreType.DMA,
            pltpu.SemaphoreType.DMA,
            pltpu.SemaphoreType.DMA,
        ],
        compiler_params=pltpu.CompilerParams(use_tc_tiling_on_sc=False),
    )
    def gather(tbl_hbm, idx_hbm, out_hbm, idx_v, row_v, g0, g1, s0, s1):
        c = lax.axis_index("c")
        s = lax.axis_index("s")
        wid = s * NC + c
        gbase = (start_row + wid * rlo + jnp.minimum(wid, rem)) * CHUNK
        lbase = (wid * rlo + jnp.minimum(wid, rem)) * CHUNK
        gsem = (g0, g1)
        ssem = (s0, s1)
        pltpu.sync_copy(idx_hbm.at[pl.ds(gbase, ew)], idx_v.at[pl.ds(0, ew)])
        if rem:
            pl.when(wid < rem)(lambda: pltpu.sync_copy(
                idx_hbm.at[pl.ds(gbase + ew, CHUNK)],
                idx_v.at[pl.ds(ew, CHUNK)]))

        def issue_gather(g, b):
            return pltpu.async_copy(
                tbl_hbm.at[idx_v.at[pl.ds(g * eg, eg)]], row_v.at[b], gsem[b])

        def issue_store(g, b):
            return pltpu.async_copy(
                row_v.at[b], out_hbm.at[pl.ds(lbase + g * eg, eg)], ssem[b])

        stores = [None] * supg
        pending = issue_gather(0, 0)
        for g in range(supg):
            b = g % 2
            pending.wait()
            stores[g] = issue_store(g, b)
            if g + 1 < supg:
                if g >= 1:
                    stores[g - 1].wait()
                pending = issue_gather(g + 1, (g + 1) % 2)
        stores[supg - 2].wait()
        stores[supg - 1].wait()

        if rem:

            def extra():
                pltpu.async_copy(tbl_hbm.at[idx_v.at[pl.ds(ew, CHUNK)]],
                                 row_v.at[0, pl.ds(0, CHUNK)], g0).wait()
                pltpu.async_copy(row_v.at[0, pl.ds(0, CHUNK)],
                                 out_hbm.at[pl.ds(lbase + ew, CHUNK)],
                                 s0).wait()

            pl.when(wid < rem)(extra)

    return gather


_gathers = [_make_gather(sr, rlo, rem, 5) for sr, rlo, rem in _CHUNKS]


KS = 3
SUPS = RLO // KS
ES = KS * CHUNK


@functools.partial(
    pl.kernel,
    out_type=jax.ShapeDtypeStruct((NC, N, TPW), jnp.float32),
    mesh=_sc_mesh,
    scratch_types=[
        pltpu.VMEM((2, ES), jnp.int32),
        pltpu.VMEM((2, ES, TPW), jnp.float32),
        pltpu.VMEM_SHARED((N, TPW), jnp.float32),
        pltpu.SemaphoreType.DMA,
        pltpu.SemaphoreType.DMA,
        pltpu.SemaphoreType.DMA,
        pltpu.SemaphoreType.DMA,
    ],
    compiler_params=pltpu.CompilerParams(use_tc_tiling_on_sc=False),
)
def _sc_scatter(tp_hbm, idx_hbm, zeros_hbm, out_hbm, idx_v, src_v, acc_sh,
                g0, g1, s0, s1):
    c = lax.axis_index("c")
    s = lax.axis_index("s")
    wid = s * NC + c
    base = (wid * RLO + jnp.minimum(wid, REM)) * CHUNK
    lsem = (g0, g1)
    asem = (s0, s1)
    pl.when(s < NSUB - 1)(lambda: pltpu.sync_copy(
        zeros_hbm, acc_sh.at[pl.ds(s * NPT, NPT)]))
    pl.when(s == NSUB - 1)(lambda: pltpu.sync_copy(
        zeros_hbm.at[pl.ds(0, NPT_LAST)], acc_sh.at[pl.ds(15 * NPT, NPT_LAST)]))
    plsc.subcore_barrier()

    def issue_load(g, b):
        d1 = pltpu.async_copy(idx_hbm.at[pl.ds(base + g * ES, ES)],
                              idx_v.at[b], lsem[b])
        d2 = pltpu.async_copy(tp_hbm.at[pl.ds(base + g * ES, ES)],
                              src_v.at[b], lsem[b])
        return (d1, d2)

    def issue_scat(g, b):
        return pltpu.async_copy(src_v.at[b], acc_sh.at[idx_v.at[b]], asem[b],
                                add=True)

    scats = [None] * SUPS
    pending = issue_load(0, 0)
    for g in range(SUPS):
        b = g % 2
        pending[0].wait()
        pending[1].wait()
        scats[g] = issue_scat(g, b)
        if g + 1 < SUPS:
            if g >= 1:
                scats[g - 1].wait()
            pending = issue_load(g + 1, (g + 1) % 2)
    scats[SUPS - 2].wait()
    scats[SUPS - 1].wait()

    def extra():
        r = base + RLO * CHUNK
        pltpu.async_copy(idx_hbm.at[pl.ds(r, CHUNK)],
                         idx_v.at[0, pl.ds(0, CHUNK)], g0).wait()
        pltpu.async_copy(tp_hbm.at[pl.ds(r, CHUNK)],
                         src_v.at[0, pl.ds(0, CHUNK)], g0).wait()
        pltpu.async_copy(src_v.at[0, pl.ds(0, CHUNK)],
                         acc_sh.at[idx_v.at[0, pl.ds(0, CHUNK)]], s0,
                         add=True).wait()

    pl.when(wid < REM)(extra)
    plsc.subcore_barrier()
    pl.when(s < NSUB - 1)(lambda: pltpu.sync_copy(
        acc_sh.at[pl.ds(s * NPT, NPT)], out_hbm.at[c, pl.ds(s * NPT, NPT)]))
    pl.when(s == NSUB - 1)(lambda: pltpu.sync_copy(
        acc_sh.at[pl.ds(15 * NPT, NPT_LAST)],
        out_hbm.at[c, pl.ds(15 * NPT, NPT_LAST)]))


TILE = 3200


def _edge_body_first(ea_ref, sh_ref, x_ref, w1_ref, b1_ref, w2_ref, b2_ref,
                     p_ref, s_ref, q_ref, out_ref):
    h = jnp.maximum(
        jnp.dot(ea_ref[...], w1_ref[...], preferred_element_type=jnp.float32)
        + b1_ref[...], 0.0)
    w = jnp.dot(h, w2_ref[...], preferred_element_type=jnp.float32) + b2_ref[...]
    y = jnp.dot(x_ref[...], p_ref[...], preferred_element_type=jnp.float32) * w
    pre = jnp.dot(y, s_ref[...], preferred_element_type=jnp.float32)
    shm = jnp.dot(sh_ref[...], q_ref[...], preferred_element_type=jnp.float32)
    tp = pre * shm * ALPHA
    ones = jnp.ones((TILE, 1), jnp.float32)
    zer = jnp.zeros((TILE, 3), jnp.float32)
    out_ref[...] = jnp.concatenate([tp, ones, zer], axis=1)


def _edge_body_chained(tpin_ref, ea_ref, sh_ref, x_ref, w1_ref, b1_ref,
                       w2_ref, b2_ref, p_ref, s_ref, q_ref, out_ref):
    del tpin_ref
    _edge_body_first(ea_ref, sh_ref, x_ref, w1_ref, b1_ref, w2_ref, b2_ref,
                     p_ref, s_ref, q_ref, out_ref)


def _make_edge_call(start_row, rlo, rem, first):
    ec = (32 * rlo + rem) * CHUNK
    off = start_row * CHUNK // TILE
    specs = [
        pl.BlockSpec((TILE, NEF), lambda i: (i + off, 0)),
        pl.BlockSpec((TILE, SH_DIM), lambda i: (i + off, 0)),
        pl.BlockSpec((TILE, NS), lambda i: (i, 0)),
        pl.BlockSpec((NEF, NEF), lambda i: (0, 0)),
        pl.BlockSpec((1, NEF), lambda i: (0, 0)),
        pl.BlockSpec((NEF, WN), lambda i: (0, 0)),
        pl.BlockSpec((1, WN), lambda i: (0, 0)),
        pl.BlockSpec((NS, WN), lambda i: (0, 0)),
        pl.BlockSpec((WN, 28), lambda i: (0, 0)),
        pl.BlockSpec((SH_DIM, 28), lambda i: (0, 0)),
    ]
    if first:
        body = _edge_body_first
        aliases = {}
    else:
        body = _edge_body_chained
        specs = [pl.BlockSpec(memory_space=pl.ANY)] + specs
        aliases = {0: 0}
    return pl.pallas_call(
        body,
        grid=(ec // TILE,),
        in_specs=specs,
        out_specs=pl.BlockSpec((TILE, TPW), lambda i: (i + off, 0)),
        out_shape=jax.ShapeDtypeStruct((E, TPW), jnp.float32),
        input_output_aliases=aliases,
    )


_edge_calls = [
    _make_edge_call(sr, rlo, rem, k == 0)
    for k, (sr, rlo, rem) in enumerate(_CHUNKS)
]


NT = 2000
NG = N // NT


def _stats_body(part_ref, out_ref):
    p = part_ref[0] + part_ref[1]
    cnt = jnp.maximum(p[:, 28:29], 1.0)
    o = p / cnt
    sums = jnp.sum(o, axis=0, keepdims=True)
    sq = jnp.sum(o * o, axis=0, keepdims=True)
    blk = jnp.concatenate([sums, sq], axis=0)

    @pl.when(pl.program_id(0) == 0)
    def _():
        out_ref[...] = jnp.zeros_like(out_ref)

    out_ref[...] += blk


_stats_call = pl.pallas_call(
    _stats_body,
    grid=(NG,),
    in_specs=[pl.BlockSpec((2, NT, TPW), lambda i: (0, i, 0))],
    out_specs=pl.BlockSpec((2, TPW), lambda i: (0, 0)),
    out_shape=jax.ShapeDtypeStruct((2, TPW), jnp.float32),
)


def _bn_body(part_ref, stats_ref, bnws_ref, bnwv_ref, bias_ref, r_ref, rt_ref,
             out_ref):
    p = part_ref[0] + part_ref[1]
    cnt = jnp.maximum(p[:, 28:29], 1.0)
    o = p[:, :28] / cnt
    sc = o[:, :NS]
    v = o[:, NS:28]
    stats = stats_ref[...] * (1.0 / N)
    m = stats[0:1, :NS]
    var = stats[1:2, :NS] - m * m
    s_bn = (sc - m) * lax.rsqrt(var + 1e-5) * bnws_ref[...] + bias_ref[...]
    vn = jnp.dot(stats[1:2, NS:28], r_ref[...],
                 preferred_element_type=jnp.float32)
    scl = bnwv_ref[...] * lax.rsqrt(vn + 1e-5)
    v_bn = v * jnp.dot(scl, rt_ref[...], preferred_element_type=jnp.float32)
    out_ref[...] = jnp.concatenate([s_bn, v_bn], axis=1)


_bn_call = pl.pallas_call(
    _bn_body,
    grid=(NG,),
    in_specs=[
        pl.BlockSpec((2, NT, TPW), lambda i: (0, i, 0)),
        pl.BlockSpec((2, TPW), lambda i: (0, 0)),
        pl.BlockSpec((1, NS), lambda i: (0, 0)),
        pl.BlockSpec((1, NV), lambda i: (0, 0)),
        pl.BlockSpec((1, NS), lambda i: (0, 0)),
        pl.BlockSpec((12, NV), lambda i: (0, 0)),
        pl.BlockSpec((NV, 12), lambda i: (0, 0)),
    ],
    out_specs=pl.BlockSpec((NT, 28), lambda i: (i, 0)),
    out_shape=jax.ShapeDtypeStruct((N, 28), jnp.float32),
)


def kernel(node_attr, edge_index, edge_attr, edge_sh, fc_w1, fc_b1, fc_w2,
           fc_b2, bn_weight, bn_bias):
    dst1 = edge_index[1]
    src1 = edge_index[0]
    xs = [g(node_attr, dst1) for g in _gathers]
    wargs = (fc_w1, fc_b1.reshape(1, NEF), fc_w2, fc_b2.reshape(1, WN),
             jnp.asarray(_P_np), jnp.asarray(_S_np), jnp.asarray(_Q_np))
    tp = _edge_calls[0](edge_attr, edge_sh, xs[0], *wargs)
    tp = _edge_calls[1](tp, edge_attr, edge_sh, xs[1], *wargs)
    tp = _edge_calls[2](tp, edge_attr, edge_sh, xs[2], *wargs)
    parts = _sc_scatter(tp, src1, jnp.zeros((NPT, TPW), jnp.float32))
    stats = _stats_call(parts)
    out = _bn_call(parts, stats, bn_weight[:NS].reshape(1, NS),
                   bn_weight[NS:].reshape(1, NV), bn_bias.reshape(1, NS),
                   jnp.asarray(_R_np), jnp.asarray(_RT_np))
    return out
